# baseline (speedup 1.0000x reference)
import jax
import jax.numpy as jnp
from jax import lax
from jax.experimental import pallas as pl
from jax.experimental.pallas import tpu as pltpu
from jax.experimental.pallas import tpu_sc as plsc

N = 160000
D = 256
NUM_BAGS = 10000

_CH = 128
_NCHUNK = N // _CH
_NW = 32
_BAGS_PAD = 10240
_ZSL = _BAGS_PAD // 16

_ROWS_PER_BLK = 8192
_BPC = _ROWS_PER_BLK // _CH

_S1_CHUNKS = 768
_S2_CHUNKS = _NCHUNK - _S1_CHUNKS
_S1_ROWS = _S1_CHUNKS * _CH
_S2_ROWS = N - _S1_ROWS


def _proj_body(x_ref, w_ref, y0_ref, y1_ref):
    y = lax.dot_general(
        w_ref[...].astype(jnp.bfloat16), x_ref[...].astype(jnp.bfloat16),
        dimension_numbers=(((1,), (1,)), ((), ())),
        preferred_element_type=jnp.float32)
    y0_ref[...] = y[0:1, :]
    y1_ref[...] = y[1:2, :]


def _project(x, w, blk0, nblk, ylen):
    return pl.pallas_call(
        _proj_body,
        grid=(nblk,),
        in_specs=[
            pl.BlockSpec((_ROWS_PER_BLK, D), lambda i: (i + blk0, 0)),
            pl.BlockSpec((2, D), lambda i: (0, 0)),
        ],
        out_specs=[
            pl.BlockSpec((1, _ROWS_PER_BLK), lambda i: (0, i)),
            pl.BlockSpec((1, _ROWS_PER_BLK), lambda i: (0, i)),
        ],
        out_shape=[
            jax.ShapeDtypeStruct((1, ylen), jnp.float32),
            jax.ShapeDtypeStruct((1, ylen), jnp.float32),
        ],
    )(x, w)


def _make_segsum(cpt, g0, scount, ylen):
    def body(idx_hbm, y0_hbm, y1_hbm, s0_hbm, s1_hbm, cnt_hbm,
             idx_v, y0_v, y1_v, ones_v, zeros_v, acc0, acc1, accc, sem0):
        cid = lax.axis_index("c")
        sid = lax.axis_index("s")
        wid = sid * 2 + cid

        def zstore(i, _):
            zeros_v[pl.ds(i * 16, 16)] = jnp.zeros((16,), jnp.float32)
            return 0
        lax.fori_loop(0, _ZSL // 16, zstore, 0)
        for a in (acc0, acc1, accc):
            pltpu.sync_copy(zeros_v, a.at[pl.ds(sid * _ZSL, _ZSL)])

        def ostore(i, _):
            ones_v[pl.ds(i * 16, 16)] = jnp.ones((16,), jnp.float32)
            return 0
        lax.fori_loop(0, _CH // 16, ostore, 0)

        base = wid * cpt
        nc = jnp.maximum(jnp.minimum(cpt, scount - base), 0)
        vstart = pl.multiple_of(
            jnp.maximum(jnp.minimum(base * _CH, ylen - cpt * _CH), 0), _CH)
        voff = base * _CH - vstart
        pltpu.sync_copy(idx_hbm.at[pl.ds(g0 + base, cpt)], idx_v)
        pltpu.sync_copy(y0_hbm.at[0, pl.ds(vstart, cpt * _CH)], y0_v)
        pltpu.sync_copy(y1_hbm.at[0, pl.ds(vstart, cpt * _CH)], y1_v)

        plsc.subcore_barrier()

        def fire(j, _):
            irow = idx_v.at[j]
            pltpu.async_copy(y0_v.at[pl.ds(voff + j * _CH, _CH)],
                             acc0.at[irow], sem0, add=True)
            pltpu.async_copy(y1_v.at[pl.ds(voff + j * _CH, _CH)],
                             acc1.at[irow], sem0, add=True)
            pltpu.async_copy(ones_v, accc.at[irow], sem0, add=True)
            return 0
        lax.fori_loop(0, nc, fire, 0)

        def drain(j, _):
            for _k in range(3):
                pltpu.make_async_copy(
                    y0_hbm.at[0, pl.ds(0, _CH)], y0_v.at[pl.ds(0, _CH)],
                    sem0).wait()
            return 0
        lax.fori_loop(0, nc, drain, 0)

        plsc.subcore_barrier()

        @pl.when(sid == 0)
        def _():
            pltpu.sync_copy(acc0,
                            s0_hbm.at[0, pl.ds(cid * _BAGS_PAD, _BAGS_PAD)])
            pltpu.sync_copy(acc1,
                            s1_hbm.at[0, pl.ds(cid * _BAGS_PAD, _BAGS_PAD)])
            pltpu.sync_copy(accc,
                            cnt_hbm.at[0, pl.ds(cid * _BAGS_PAD, _BAGS_PAD)])

    f32 = jnp.float32
    return pl.kernel(
        body,
        out_type=[
            jax.ShapeDtypeStruct((1, 2 * _BAGS_PAD), f32),
            jax.ShapeDtypeStruct((1, 2 * _BAGS_PAD), f32),
            jax.ShapeDtypeStruct((1, 2 * _BAGS_PAD), f32),
        ],
        mesh=plsc.VectorSubcoreMesh(
            core_axis_name="c", subcore_axis_name="s",
            num_cores=2, num_subcores=16),
        scratch_types=[
            pltpu.VMEM((cpt, _CH), jnp.int32),
            pltpu.VMEM((cpt * _CH,), f32),
            pltpu.VMEM((cpt * _CH,), f32),
            pltpu.VMEM((_CH,), f32),
            pltpu.VMEM((_ZSL,), f32),
            pltpu.VMEM_SHARED((_BAGS_PAD,), f32),
            pltpu.VMEM_SHARED((_BAGS_PAD,), f32),
            pltpu.VMEM_SHARED((_BAGS_PAD,), f32),
            pltpu.SemaphoreType.DMA,
        ],
    )


_segsum1 = _make_segsum(24, 0, _S1_CHUNKS, _S1_ROWS)
_segsum2 = _make_segsum(16, _S1_CHUNKS, _S2_CHUNKS, _S2_ROWS)


def _final_body(a0_ref, a1_ref, ac_ref, b0_ref, b1_ref, bc_ref, b_ref,
                out_ref):
    def tot(r):
        return r[0:1, :_BAGS_PAD] + r[0:1, _BAGS_PAD:]
    t0 = tot(a0_ref) + tot(b0_ref)
    t1 = tot(a1_ref) + tot(b1_ref)
    c = tot(ac_ref) + tot(bc_ref)
    denom = jnp.maximum(c, 1.0)
    l0 = t0 / denom + b_ref[0]
    l1 = t1 / denom + b_ref[1]
    m = jnp.maximum(l0, l1)
    e0 = jnp.exp(l0 - m)
    e1 = jnp.exp(l1 - m)
    s = e0 + e1
    out_ref[0:1, :] = e0 / s
    out_ref[1:2, :] = e1 / s


def _finalize(sa, sb, b):
    return pl.pallas_call(
        _final_body,
        in_specs=[pl.BlockSpec(memory_space=pltpu.VMEM)] * 6
        + [pl.BlockSpec(memory_space=pltpu.SMEM)],
        out_shape=jax.ShapeDtypeStruct((2, _BAGS_PAD), jnp.float32),
    )(sa[0], sa[1], sa[2], sb[0], sb[1], sb[2], b)


@jax.jit
def kernel(bag_encoding, batch_indices, W, b):
    idx = jnp.concatenate(
        [batch_indices.astype(jnp.int32),
         jnp.zeros((_NW * 40 * _CH - N,), jnp.int32)]
    ).reshape(_NW * 40, _CH)
    ya = _project(bag_encoding, W, 0, _S1_CHUNKS // _BPC, _S1_ROWS)
    sa = _segsum1(idx, ya[0], ya[1])
    yb = _project(bag_encoding, W, _S1_CHUNKS // _BPC,
                  (_S2_CHUNKS + _BPC - 1) // _BPC, _S2_ROWS)
    sb = _segsum2(idx, yb[0], yb[1])
    probs = _finalize(sa, sb, b)
    return probs.T[:NUM_BAGS, :]


# speedup vs baseline: 1.0231x; 1.0231x over previous
import jax
import jax.numpy as jnp
from jax import lax
from jax.experimental import pallas as pl
from jax.experimental.pallas import tpu as pltpu
from jax.experimental.pallas import tpu_sc as plsc

N = 160000
D = 256
NUM_BAGS = 10000

_CH = 128
_NCHUNK = N // _CH
_NW = 32
_BAGS_PAD = 10240
_ZSL = _BAGS_PAD // 16

_ROWS_PER_BLK = 8192
_BPC = _ROWS_PER_BLK // _CH

_SLICES = ((0, 512, 16), (512, 512, 16), (1024, 226, 8))


def _proj_body(x_ref, w_ref, idx_ref, y0_ref, y1_ref, idxp_ref):
    y = lax.dot_general(
        w_ref[...].astype(jnp.bfloat16), x_ref[...].astype(jnp.bfloat16),
        dimension_numbers=(((1,), (1,)), ((), ())),
        preferred_element_type=jnp.float32)
    y0_ref[...] = y[0:1, :]
    y1_ref[...] = y[1:2, :]
    idxp_ref[...] = idx_ref[...]


def _project(x, w, idx2d, blk0, nblk, ylen, ichunks):
    return pl.pallas_call(
        _proj_body,
        grid=(nblk,),
        in_specs=[
            pl.BlockSpec((_ROWS_PER_BLK, D), lambda i: (i + blk0, 0)),
            pl.BlockSpec((2, D), lambda i: (0, 0)),
            pl.BlockSpec((_BPC, _CH), lambda i: (i + blk0, 0)),
        ],
        out_specs=[
            pl.BlockSpec((1, _ROWS_PER_BLK), lambda i: (0, i)),
            pl.BlockSpec((1, _ROWS_PER_BLK), lambda i: (0, i)),
            pl.BlockSpec((_BPC, _CH), lambda i: (i, 0)),
        ],
        out_shape=[
            jax.ShapeDtypeStruct((1, ylen), jnp.float32),
            jax.ShapeDtypeStruct((1, ylen), jnp.float32),
            jax.ShapeDtypeStruct((ichunks, _CH), jnp.int32),
        ],
    )(x, w, idx2d)


def _make_segsum(cpt, scount, ylen):
    def body(idx_hbm, y0_hbm, y1_hbm, s0_hbm, s1_hbm, cnt_hbm,
             idx_v, y0_v, y1_v, ones_v, zeros_v, acc0, acc1, accc, sem0):
        cid = lax.axis_index("c")
        sid = lax.axis_index("s")
        wid = sid * 2 + cid

        base = wid * cpt
        nc = jnp.maximum(jnp.minimum(cpt, scount - base), 0)
        vstart = pl.multiple_of(
            jnp.maximum(jnp.minimum(base * _CH, ylen - cpt * _CH), 0), _CH)
        voff = base * _CH - vstart
        st0 = pltpu.async_copy(idx_hbm.at[pl.ds(base, cpt)], idx_v, sem0)
        st1 = pltpu.async_copy(y0_hbm.at[0, pl.ds(vstart, cpt * _CH)],
                               y0_v, sem0)
        st2 = pltpu.async_copy(y1_hbm.at[0, pl.ds(vstart, cpt * _CH)],
                               y1_v, sem0)

        def zstore(i, _):
            zeros_v[pl.ds(i * 16, 16)] = jnp.zeros((16,), jnp.float32)
            return 0
        lax.fori_loop(0, _ZSL // 16, zstore, 0)
        for a in (acc0, acc1, accc):
            pltpu.sync_copy(zeros_v, a.at[pl.ds(sid * _ZSL, _ZSL)])

        def ostore(i, _):
            ones_v[pl.ds(i * 16, 16)] = jnp.ones((16,), jnp.float32)
            return 0
        lax.fori_loop(0, _CH // 16, ostore, 0)

        st0.wait()
        st1.wait()
        st2.wait()
        plsc.subcore_barrier()

        def fire(j, _):
            irow = idx_v.at[j]
            pltpu.async_copy(y0_v.at[pl.ds(voff + j * _CH, _CH)],
                             acc0.at[irow], sem0, add=True)
            pltpu.async_copy(y1_v.at[pl.ds(voff + j * _CH, _CH)],
                             acc1.at[irow], sem0, add=True)
            pltpu.async_copy(ones_v, accc.at[irow], sem0, add=True)
            return 0
        lax.fori_loop(0, nc, fire, 0)

        def drain(j, _):
            for _k in range(3):
                pltpu.make_async_copy(
                    y0_hbm.at[0, pl.ds(0, _CH)], y0_v.at[pl.ds(0, _CH)],
                    sem0).wait()
            return 0
        lax.fori_loop(0, nc, drain, 0)

        plsc.subcore_barrier()

        @pl.when(sid == 0)
        def _():
            pltpu.sync_copy(acc0,
                            s0_hbm.at[0, pl.ds(cid * _BAGS_PAD, _BAGS_PAD)])
            pltpu.sync_copy(acc1,
                            s1_hbm.at[0, pl.ds(cid * _BAGS_PAD, _BAGS_PAD)])
            pltpu.sync_copy(accc,
                            cnt_hbm.at[0, pl.ds(cid * _BAGS_PAD, _BAGS_PAD)])

    f32 = jnp.float32
    return pl.kernel(
        body,
        out_type=[
            jax.ShapeDtypeStruct((1, 2 * _BAGS_PAD), f32),
            jax.ShapeDtypeStruct((1, 2 * _BAGS_PAD), f32),
            jax.ShapeDtypeStruct((1, 2 * _BAGS_PAD), f32),
        ],
        mesh=plsc.VectorSubcoreMesh(
            core_axis_name="c", subcore_axis_name="s",
            num_cores=2, num_subcores=16),
        scratch_types=[
            pltpu.VMEM((cpt, _CH), jnp.int32),
            pltpu.VMEM((cpt * _CH,), f32),
            pltpu.VMEM((cpt * _CH,), f32),
            pltpu.VMEM((_CH,), f32),
            pltpu.VMEM((_ZSL,), f32),
            pltpu.VMEM_SHARED((_BAGS_PAD,), f32),
            pltpu.VMEM_SHARED((_BAGS_PAD,), f32),
            pltpu.VMEM_SHARED((_BAGS_PAD,), f32),
            pltpu.SemaphoreType.DMA,
        ],
    )


_segsums = [_make_segsum(cpt, cnt, min((g0 + cnt) * _CH, N) - g0 * _CH)
            for (g0, cnt, cpt) in _SLICES]


def _final_body(a0, a1, ac, b0, b1, bc, c0, c1, cc, b_ref, out_ref):
    def tot(r):
        return r[0:1, :_BAGS_PAD] + r[0:1, _BAGS_PAD:]
    t0 = tot(a0) + tot(b0) + tot(c0)
    t1 = tot(a1) + tot(b1) + tot(c1)
    c = tot(ac) + tot(bc) + tot(cc)
    denom = jnp.maximum(c, 1.0)
    l0 = t0 / denom + b_ref[0]
    l1 = t1 / denom + b_ref[1]
    m = jnp.maximum(l0, l1)
    e0 = jnp.exp(l0 - m)
    e1 = jnp.exp(l1 - m)
    s = e0 + e1
    out_ref[0:1, :] = e0 / s
    out_ref[1:2, :] = e1 / s


def _finalize(sa, sb, sc, b):
    return pl.pallas_call(
        _final_body,
        in_specs=[pl.BlockSpec(memory_space=pltpu.VMEM)] * 9
        + [pl.BlockSpec(memory_space=pltpu.SMEM)],
        out_shape=jax.ShapeDtypeStruct((2, _BAGS_PAD), jnp.float32),
    )(*sa, *sb, *sc, b)


@jax.jit
def kernel(bag_encoding, batch_indices, W, b):
    idx2d = batch_indices.astype(jnp.int32).reshape(_NCHUNK, _CH)
    parts = []
    for (g0, cnt, cpt), seg in zip(_SLICES, _segsums):
        ylen = min((g0 + cnt) * _CH, N) - g0 * _CH
        nblk = (cnt + _BPC - 1) // _BPC
        y0, y1, idxp = _project(bag_encoding, W, idx2d,
                                g0 // _BPC, nblk, ylen, nblk * _BPC)
        parts.append(seg(idxp, y0, y1))
    probs = _finalize(parts[0], parts[1], parts[2], b)
    return probs.T[:NUM_BAGS, :]


# speedup vs baseline: 1.0386x; 1.0152x over previous
import jax
import jax.numpy as jnp
from jax import lax
from jax.experimental import pallas as pl
from jax.experimental.pallas import tpu as pltpu
from jax.experimental.pallas import tpu_sc as plsc

N = 160000
D = 256
NUM_BAGS = 10000

_CH = 128
_NCHUNK = N // _CH
_NW = 32
_BAGS_PAD = 10240
_ZSL = _BAGS_PAD // 16

_ROWS_PER_BLK = 8192
_BPC = _ROWS_PER_BLK // _CH

_SLICES = ((0, 704, 22), (704, 384, 12), (1088, 162, 6))


def _proj_body(x_ref, w_ref, idx_ref, y0_ref, y1_ref, idxp_ref):
    y = lax.dot_general(
        w_ref[...].astype(jnp.bfloat16), x_ref[...].astype(jnp.bfloat16),
        dimension_numbers=(((1,), (1,)), ((), ())),
        preferred_element_type=jnp.float32)
    y0_ref[...] = y[0:1, :]
    y1_ref[...] = y[1:2, :]
    idxp_ref[...] = idx_ref[...]


def _project(x, w, idx2d, blk0, nblk, ylen, ichunks):
    return pl.pallas_call(
        _proj_body,
        grid=(nblk,),
        in_specs=[
            pl.BlockSpec((_ROWS_PER_BLK, D), lambda i: (i + blk0, 0)),
            pl.BlockSpec((2, D), lambda i: (0, 0)),
            pl.BlockSpec((_BPC, _CH), lambda i: (i + blk0, 0)),
        ],
        out_specs=[
            pl.BlockSpec((1, _ROWS_PER_BLK), lambda i: (0, i)),
            pl.BlockSpec((1, _ROWS_PER_BLK), lambda i: (0, i)),
            pl.BlockSpec((_BPC, _CH), lambda i: (i, 0)),
        ],
        out_shape=[
            jax.ShapeDtypeStruct((1, ylen), jnp.float32),
            jax.ShapeDtypeStruct((1, ylen), jnp.float32),
            jax.ShapeDtypeStruct((ichunks, _CH), jnp.int32),
        ],
    )(x, w, idx2d)


def _spad(cpt):
    return (cpt + 7 + 7) // 8 * 8


def _make_segsum(cpt, scount, ylen):
    def body(idx_hbm, y0_hbm, y1_hbm, s0_hbm, s1_hbm, cnt_hbm,
             idx_v, y0_v, y1_v, ones_v, zeros_v, acc0, acc1, accc, sem0):
        cid = lax.axis_index("c")
        sid = lax.axis_index("s")
        wid = sid * 2 + cid

        base = wid * cpt
        nc = jnp.maximum(jnp.minimum(cpt, scount - base), 0)
        vstart = pl.multiple_of(
            jnp.maximum(jnp.minimum(base * _CH, ylen - cpt * _CH), 0), _CH)
        voff = base * _CH - vstart
        ibase = pl.multiple_of((base // 8) * 8, 8)
        ioff = base - ibase
        st0 = pltpu.async_copy(idx_hbm.at[pl.ds(ibase, _spad(cpt))], idx_v,
                               sem0)
        st1 = pltpu.async_copy(y0_hbm.at[0, pl.ds(vstart, cpt * _CH)],
                               y0_v, sem0)
        st2 = pltpu.async_copy(y1_hbm.at[0, pl.ds(vstart, cpt * _CH)],
                               y1_v, sem0)

        def zstore(i, _):
            zeros_v[pl.ds(i * 16, 16)] = jnp.zeros((16,), jnp.float32)
            return 0
        lax.fori_loop(0, _ZSL // 16, zstore, 0)
        for a in (acc0, acc1, accc):
            pltpu.sync_copy(zeros_v, a.at[pl.ds(sid * _ZSL, _ZSL)])

        def ostore(i, _):
            ones_v[pl.ds(i * 16, 16)] = jnp.ones((16,), jnp.float32)
            return 0
        lax.fori_loop(0, _CH // 16, ostore, 0)

        st0.wait()
        st1.wait()
        st2.wait()
        plsc.subcore_barrier()

        def fire(j, _):
            irow = idx_v.at[ioff + j]
            pltpu.async_copy(y0_v.at[pl.ds(voff + j * _CH, _CH)],
                             acc0.at[irow], sem0, add=True)
            pltpu.async_copy(y1_v.at[pl.ds(voff + j * _CH, _CH)],
                             acc1.at[irow], sem0, add=True)
            pltpu.async_copy(ones_v, accc.at[irow], sem0, add=True)
            return 0
        lax.fori_loop(0, nc, fire, 0)

        def drain(j, _):
            for _k in range(3):
                pltpu.make_async_copy(
                    y0_hbm.at[0, pl.ds(0, _CH)], y0_v.at[pl.ds(0, _CH)],
                    sem0).wait()
            return 0
        lax.fori_loop(0, nc, drain, 0)

        plsc.subcore_barrier()

        @pl.when(sid == 0)
        def _():
            pltpu.sync_copy(acc0,
                            s0_hbm.at[0, pl.ds(cid * _BAGS_PAD, _BAGS_PAD)])
            pltpu.sync_copy(acc1,
                            s1_hbm.at[0, pl.ds(cid * _BAGS_PAD, _BAGS_PAD)])
            pltpu.sync_copy(accc,
                            cnt_hbm.at[0, pl.ds(cid * _BAGS_PAD, _BAGS_PAD)])

    f32 = jnp.float32
    return pl.kernel(
        body,
        out_type=[
            jax.ShapeDtypeStruct((1, 2 * _BAGS_PAD), f32),
            jax.ShapeDtypeStruct((1, 2 * _BAGS_PAD), f32),
            jax.ShapeDtypeStruct((1, 2 * _BAGS_PAD), f32),
        ],
        mesh=plsc.VectorSubcoreMesh(
            core_axis_name="c", subcore_axis_name="s",
            num_cores=2, num_subcores=16),
        scratch_types=[
            pltpu.VMEM((_spad(cpt), _CH), jnp.int32),
            pltpu.VMEM((cpt * _CH,), f32),
            pltpu.VMEM((cpt * _CH,), f32),
            pltpu.VMEM((_CH,), f32),
            pltpu.VMEM((_ZSL,), f32),
            pltpu.VMEM_SHARED((_BAGS_PAD,), f32),
            pltpu.VMEM_SHARED((_BAGS_PAD,), f32),
            pltpu.VMEM_SHARED((_BAGS_PAD,), f32),
            pltpu.SemaphoreType.DMA,
        ],
    )


_segsums = [_make_segsum(cpt, cnt, min((g0 + cnt) * _CH, N) - g0 * _CH)
            for (g0, cnt, cpt) in _SLICES]


def _final_body(a0, a1, ac, b0, b1, bc, c0, c1, cc, b_ref, out_ref):
    def tot(r):
        return r[0:1, :_BAGS_PAD] + r[0:1, _BAGS_PAD:]
    t0 = tot(a0) + tot(b0) + tot(c0)
    t1 = tot(a1) + tot(b1) + tot(c1)
    c = tot(ac) + tot(bc) + tot(cc)
    denom = jnp.maximum(c, 1.0)
    l0 = t0 / denom + b_ref[0]
    l1 = t1 / denom + b_ref[1]
    m = jnp.maximum(l0, l1)
    e0 = jnp.exp(l0 - m)
    e1 = jnp.exp(l1 - m)
    s = e0 + e1
    out_ref[0:1, :] = e0 / s
    out_ref[1:2, :] = e1 / s


def _finalize(sa, sb, sc, b):
    return pl.pallas_call(
        _final_body,
        in_specs=[pl.BlockSpec(memory_space=pltpu.VMEM)] * 9
        + [pl.BlockSpec(memory_space=pltpu.SMEM)],
        out_shape=jax.ShapeDtypeStruct((2, _BAGS_PAD), jnp.float32),
    )(*sa, *sb, *sc, b)


@jax.jit
def kernel(bag_encoding, batch_indices, W, b):
    idx2d = batch_indices.astype(jnp.int32).reshape(_NCHUNK, _CH)
    parts = []
    for (g0, cnt, cpt), seg in zip(_SLICES, _segsums):
        ylen = min((g0 + cnt) * _CH, N) - g0 * _CH
        nblk = (cnt + _BPC - 1) // _BPC
        ich = nblk * _BPC
        if ich < _NW * cpt + 8:
            ich += _BPC
        y0, y1, idxp = _project(bag_encoding, W, idx2d,
                                g0 // _BPC, nblk, ylen, ich)
        parts.append(seg(idxp, y0, y1))
    probs = _finalize(parts[0], parts[1], parts[2], b)
    return probs.T[:NUM_BAGS, :]


# speedup vs baseline: 1.0677x; 1.0280x over previous
import jax
import jax.numpy as jnp
from jax import lax
from jax.experimental import pallas as pl
from jax.experimental.pallas import tpu as pltpu
from jax.experimental.pallas import tpu_sc as plsc

N = 160000
D = 256
NUM_BAGS = 10000

_CH = 128
_NCHUNK = N // _CH
_NW = 32
_BAGS_PAD = 10240
_ZSL = _BAGS_PAD // 16

_ROWS_PER_BLK = 8192
_BPC = _ROWS_PER_BLK // _CH

_SLICES = ((0, 704, 22), (704, 384, 12), (1088, 162, 6))


def _proj_body(x_ref, w_ref, idx_ref, y0_ref, y1_ref, idxp_ref):
    y = lax.dot_general(
        w_ref[...].astype(jnp.bfloat16), x_ref[...].astype(jnp.bfloat16),
        dimension_numbers=(((1,), (1,)), ((), ())),
        preferred_element_type=jnp.float32)
    y0_ref[...] = y[0:1, :]
    y1_ref[...] = y[1:2, :]
    idxp_ref[...] = idx_ref[...]


def _project(x, w, idx2d, blk0, nblk, ylen, ichunks):
    return pl.pallas_call(
        _proj_body,
        grid=(nblk,),
        in_specs=[
            pl.BlockSpec((_ROWS_PER_BLK, D), lambda i: (i + blk0, 0)),
            pl.BlockSpec((2, D), lambda i: (0, 0)),
            pl.BlockSpec((_BPC, _CH), lambda i: (i + blk0, 0)),
        ],
        out_specs=[
            pl.BlockSpec((1, _ROWS_PER_BLK), lambda i: (0, i)),
            pl.BlockSpec((1, _ROWS_PER_BLK), lambda i: (0, i)),
            pl.BlockSpec((_BPC, _CH), lambda i: (i, 0)),
        ],
        out_shape=[
            jax.ShapeDtypeStruct((1, ylen), jnp.float32),
            jax.ShapeDtypeStruct((1, ylen), jnp.float32),
            jax.ShapeDtypeStruct((ichunks, _CH), jnp.int32),
        ],
    )(x, w, idx2d)


def _spad(cpt):
    return (cpt + 7 + 7) // 8 * 8


def _make_segsum(cpt, scount, ylen):
    def body(idx_hbm, y0_hbm, y1_hbm, s0_hbm, s1_hbm, cnt_hbm,
             idx_v, y0_v, y1_v, ones_v, zeros_v, acc0, acc1, accc, sem0):
        cid = lax.axis_index("c")
        sid = lax.axis_index("s")
        wid = sid * 2 + cid

        base = wid * cpt
        nc = jnp.maximum(jnp.minimum(cpt, scount - base), 0)
        vstart = pl.multiple_of(
            jnp.maximum(jnp.minimum(base * _CH, ylen - cpt * _CH), 0), _CH)
        voff = base * _CH - vstart
        ibase = pl.multiple_of((base // 8) * 8, 8)
        ioff = base - ibase
        st0 = pltpu.async_copy(idx_hbm.at[pl.ds(ibase, _spad(cpt))], idx_v,
                               sem0)
        st1 = pltpu.async_copy(y0_hbm.at[0, pl.ds(vstart, cpt * _CH)],
                               y0_v, sem0)
        st2 = pltpu.async_copy(y1_hbm.at[0, pl.ds(vstart, cpt * _CH)],
                               y1_v, sem0)

        def zstore(i, _):
            zeros_v[pl.ds(i * 16, 16)] = jnp.zeros((16,), jnp.float32)
            return 0
        lax.fori_loop(0, _ZSL // 16, zstore, 0)
        for a in (acc0, acc1, accc):
            pltpu.sync_copy(zeros_v, a.at[pl.ds(sid * _ZSL, _ZSL)])

        def ostore(i, _):
            ones_v[pl.ds(i * 16, 16)] = jnp.ones((16,), jnp.float32)
            return 0
        lax.fori_loop(0, _CH // 16, ostore, 0)

        st0.wait()
        st1.wait()
        st2.wait()
        plsc.subcore_barrier()

        def fire(j, _):
            irow = idx_v.at[ioff + j]
            pltpu.async_copy(y0_v.at[pl.ds(voff + j * _CH, _CH)],
                             acc0.at[irow], sem0, add=True)
            pltpu.async_copy(y1_v.at[pl.ds(voff + j * _CH, _CH)],
                             acc1.at[irow], sem0, add=True)
            pltpu.async_copy(ones_v, accc.at[irow], sem0, add=True)
            return 0
        lax.fori_loop(0, nc, fire, 0)

        def drain(j, _):
            for _k in range(3):
                pltpu.make_async_copy(
                    y0_hbm.at[0, pl.ds(0, _CH)], y0_v.at[pl.ds(0, _CH)],
                    sem0).wait()
            return 0
        lax.fori_loop(0, nc, drain, 0)

        plsc.subcore_barrier()

        @pl.when(sid == 0)
        def _():
            e0 = pltpu.async_copy(
                acc0, s0_hbm.at[0, pl.ds(cid * _BAGS_PAD, _BAGS_PAD)], sem0)
            e1 = pltpu.async_copy(
                acc1, s1_hbm.at[0, pl.ds(cid * _BAGS_PAD, _BAGS_PAD)], sem0)
            e2 = pltpu.async_copy(
                accc, cnt_hbm.at[0, pl.ds(cid * _BAGS_PAD, _BAGS_PAD)], sem0)
            e0.wait()
            e1.wait()
            e2.wait()

    f32 = jnp.float32
    return pl.kernel(
        body,
        out_type=[
            jax.ShapeDtypeStruct((1, 2 * _BAGS_PAD), f32),
            jax.ShapeDtypeStruct((1, 2 * _BAGS_PAD), f32),
            jax.ShapeDtypeStruct((1, 2 * _BAGS_PAD), f32),
        ],
        mesh=plsc.VectorSubcoreMesh(
            core_axis_name="c", subcore_axis_name="s",
            num_cores=2, num_subcores=16),
        scratch_types=[
            pltpu.VMEM((_spad(cpt), _CH), jnp.int32),
            pltpu.VMEM((cpt * _CH,), f32),
            pltpu.VMEM((cpt * _CH,), f32),
            pltpu.VMEM((_CH,), f32),
            pltpu.VMEM((_ZSL,), f32),
            pltpu.VMEM_SHARED((_BAGS_PAD,), f32),
            pltpu.VMEM_SHARED((_BAGS_PAD,), f32),
            pltpu.VMEM_SHARED((_BAGS_PAD,), f32),
            pltpu.SemaphoreType.DMA,
        ],
    )


_segsums = [_make_segsum(cpt, cnt, min((g0 + cnt) * _CH, N) - g0 * _CH)
            for (g0, cnt, cpt) in _SLICES]


def _final_body(a0, a1, ac, b0, b1, bc, c0, c1, cc, b_ref, out_ref):
    def tot(r):
        return r[0:1, :_BAGS_PAD] + r[0:1, _BAGS_PAD:]
    t0 = tot(a0) + tot(b0) + tot(c0)
    t1 = tot(a1) + tot(b1) + tot(c1)
    c = tot(ac) + tot(bc) + tot(cc)
    denom = jnp.maximum(c, 1.0)
    l0 = t0 / denom + b_ref[0]
    l1 = t1 / denom + b_ref[1]
    m = jnp.maximum(l0, l1)
    e0 = jnp.exp(l0 - m)
    e1 = jnp.exp(l1 - m)
    s = e0 + e1
    out_ref[0:1, :] = e0 / s
    out_ref[1:2, :] = e1 / s


def _finalize(sa, sb, sc, b):
    return pl.pallas_call(
        _final_body,
        in_specs=[pl.BlockSpec(memory_space=pltpu.VMEM)] * 9
        + [pl.BlockSpec(memory_space=pltpu.SMEM)],
        out_shape=jax.ShapeDtypeStruct((2, _BAGS_PAD), jnp.float32),
    )(*sa, *sb, *sc, b)


@jax.jit
def kernel(bag_encoding, batch_indices, W, b):
    idx2d = batch_indices.astype(jnp.int32).reshape(_NCHUNK, _CH)
    parts = []
    for (g0, cnt, cpt), seg in zip(_SLICES, _segsums):
        ylen = min((g0 + cnt) * _CH, N) - g0 * _CH
        nblk = (cnt + _BPC - 1) // _BPC
        ich = nblk * _BPC
        if ich < _NW * cpt + 8:
            ich += _BPC
        y0, y1, idxp = _project(bag_encoding, W, idx2d,
                                g0 // _BPC, nblk, ylen, ich)
        parts.append(seg(idxp, y0, y1))
    probs = _finalize(parts[0], parts[1], parts[2], b)
    return probs.T[:NUM_BAGS, :]
